# baseline (speedup 1.0000x reference)
import numpy as np
import jax
import jax.numpy as jnp
from jax.experimental import pallas as pl
from jax.experimental.pallas import tpu as pltpu

_K_MAX_L = [128, 64, 32]
_L_MAX = 2
_IRREPS_IN = [(0, 1), (1, 1), (2, 1)]
_N_ITER = 2


def _cg_tables(l_max):
    cgs = {}
    for l1 in range(l_max + 1):
        for l2 in range(l_max + 1):
            entries = []
            norms = []
            off = 0
            for L in range(abs(l1 - l2), min(l1 + l2, l_max) + 1):
                norms.append(
                    float(1.0 / np.sqrt((2 * l1 + 1) * (2 * l2 + 1) * (2 * L + 1))))
                for M in range(2 * L + 1):
                    MU = M - L
                    for m1 in range(2 * l1 + 1):
                        mu1 = m1 - l1
                        mu2 = MU - mu1
                        if -l2 <= mu2 <= l2:
                            m2 = mu2 + l2
                            entries.append((m1, m2, off + M, (-1) ** mu1))
                off += 2 * L + 1
            cgs[(l1, l2)] = (entries, off, norms)
    return cgs


_CGS = _cg_tables(_L_MAX)


def _sim_structure():
    cur = list(_IRREPS_IN)
    orders = []
    for _ in range(_N_ITER):
        order = []
        for (l1, s1) in cur:
            for (l2, s2) in cur:
                for L in range(abs(l1 - l2), min(l1 + l2, _L_MAX) + 1):
                    S = s1 * s2 * (-1) ** (l1 + l2 + L)
                    if (L, S) not in order:
                        order.append((L, S))
        orders.append(order)
        cur = order
    return orders


_ORDERS = _sim_structure()
_FINAL_ORDER = _ORDERS[-1]
_W_KEYS = [(0, (0, 1)), (0, (1, -1)), (0, (1, 1)), (0, (2, -1)), (0, (2, 1)),
           (1, (0, -1)), (1, (0, 1)), (1, (1, -1)), (1, (1, 1)),
           (1, (2, -1)), (1, (2, 1))]


def _cg_apply(cur, Wd, bd):
    gscaled = {}
    for kk, (w, tlist) in cur.items():
        l = kk[0]
        gscaled[kk] = [t * (1.0 + 0.1 * (m - l)) for m, t in enumerate(tlist)]

    results = {}
    for k1, (w1, t1) in cur.items():
        l1, s1 = k1
        for k2, (w2, _) in cur.items():
            l2, s2 = k2
            t2g = gscaled[k2]
            ms = min(w1, w2)
            entries, size_out, norms = _CGS[(l1, l2)]
            res = [None] * size_out

            def _accum(io, p, sgn):
                if res[io] is None:
                    res[io] = p if sgn > 0 else -p
                else:
                    res[io] = res[io] + p if sgn > 0 else res[io] - p

            if k1 == k2:
                byprod = {}
                for ia, ib, io, sgn in entries:
                    byprod.setdefault((ia, ib), []).append(io)
                for (ia, ib), ios in byprod.items():
                    if ia == ib:
                        p = t1[ia][:ms, :] * t2g[ia][:ms, :]
                        sgn = (-1) ** (ia - l1)
                        for io in ios:
                            _accum(io, p, sgn)
                    elif ia < ib:
                        praw = t1[ia][:ms, :] * t1[ib][:ms, :]
                        gamma = ((-1) ** (ia - l1) * (1.0 + 0.1 * (ib - l1))
                                 + (-1) ** (ib - l1) * (1.0 + 0.1 * (ia - l1)))
                        pg = praw * gamma
                        for io in ios:
                            _accum(io, pg, 1)
            else:
                p = None
                prev = None
                for ia, ib, io, sgn in sorted(entries):
                    if (ia, ib) != prev:
                        p = t1[ia][:ms, :] * t2g[ib][:ms, :]
                        prev = (ia, ib)
                    _accum(io, p, sgn)
            off = 0
            for Li, L in enumerate(
                    range(abs(l1 - l2), min(l1 + l2, _L_MAX) + 1)):
                S = s1 * s2 * (-1) ** (l1 + l2 + L)
                results.setdefault((L, S), []).append(
                    (res[off:off + 2 * L + 1], ms, norms[Li]))
                off += 2 * L + 1
    out = {}
    for (L, S), lst in results.items():
        W = Wd[(L, S)]
        b = bd[(L, S)]
        Wchs = []
        off = 0
        for (_, ms, nrm) in lst:
            Wchs.append(W[:, off:off + ms] * nrm)
            off += ms
        outm = []
        for M in range(2 * L + 1):
            acc = None
            for (arrs, ms, _), Wch in zip(lst, Wchs):
                mm = jax.lax.dot_general(
                    Wch, arrs[M], (((1,), (0,)), ((), ())),
                    preferred_element_type=jnp.float32)
                acc = mm if acc is None else acc + mm
            outm.append(acc + b)
        out[(L, S)] = (_K_MAX_L[L], outm)
    return out


def _body(f0_ref, f1_ref, f2_ref, *refs):
    Wrefs = refs[:11]
    bref = refs[11]
    orefs = refs[12:]
    Wd = [{}, {}]
    bd = [{}, {}]
    boff = 0
    for (it, ls), wr in zip(_W_KEYS, Wrefs):
        Wd[it][ls] = wr[...]
        k = _K_MAX_L[ls[0]]
        bd[it][ls] = bref[boff:boff + k, :]
        boff += k

    f0 = f0_ref[...].T
    f1 = f1_ref[...].T
    f2 = f2_ref[...].T
    cur = {
        (0, 1): (128, [f0]),
        (1, 1): (64, [f1[64 * m:64 * (m + 1), :] for m in range(3)]),
        (2, 1): (32, [f2[32 * m:32 * (m + 1), :] for m in range(5)]),
    }
    for it in range(_N_ITER):
        out = _cg_apply(cur, Wd[it], bd[it])
        new = {}
        for kk, (w, arrs) in out.items():
            if kk in cur and cur[kk][0] == w and len(cur[kk][1]) == len(arrs):
                new[kk] = (w, [a + b for a, b in zip(arrs, cur[kk][1])])
            else:
                new[kk] = (w, arrs)
        cur = new

    for oref, kk in zip(orefs, _FINAL_ORDER):
        w, arrs = cur[kk]
        for M, a in enumerate(arrs):
            oref[M, :, :] = a.T


def kernel(feat_0_1, feat_1_1, feat_2_1,
           W_it0_0_p1, W_it0_1_m1, W_it0_1_p1, W_it0_2_m1, W_it0_2_p1,
           W_it1_0_m1, W_it1_0_p1, W_it1_1_m1, W_it1_1_p1, W_it1_2_m1,
           W_it1_2_p1,
           b_it0_0_p1, b_it0_1_m1, b_it0_1_p1, b_it0_2_m1, b_it0_2_p1,
           b_it1_0_m1, b_it1_0_p1, b_it1_1_m1, b_it1_1_p1, b_it1_2_m1,
           b_it1_2_p1):
    n = feat_0_1.shape[0]
    f0 = feat_0_1.reshape(n, 128)
    f1 = feat_1_1.reshape(n, 3 * 64)
    f2 = feat_2_1.reshape(n, 5 * 32)

    Wmap = {
        (0, (0, 1)): W_it0_0_p1, (0, (1, -1)): W_it0_1_m1,
        (0, (1, 1)): W_it0_1_p1, (0, (2, -1)): W_it0_2_m1,
        (0, (2, 1)): W_it0_2_p1,
        (1, (0, -1)): W_it1_0_m1, (1, (0, 1)): W_it1_0_p1,
        (1, (1, -1)): W_it1_1_m1, (1, (1, 1)): W_it1_1_p1,
        (1, (2, -1)): W_it1_2_m1, (1, (2, 1)): W_it1_2_p1,
    }
    bmap = {
        (0, (0, 1)): b_it0_0_p1, (0, (1, -1)): b_it0_1_m1,
        (0, (1, 1)): b_it0_1_p1, (0, (2, -1)): b_it0_2_m1,
        (0, (2, 1)): b_it0_2_p1,
        (1, (0, -1)): b_it1_0_m1, (1, (0, 1)): b_it1_0_p1,
        (1, (1, -1)): b_it1_1_m1, (1, (1, 1)): b_it1_1_p1,
        (1, (2, -1)): b_it1_2_m1, (1, (2, 1)): b_it1_2_p1,
    }
    Ws = [Wmap[k] for k in _W_KEYS]
    bpacked = jnp.concatenate([bmap[k] for k in _W_KEYS]).reshape(-1, 1)

    B = min(1024, n)
    grid = (n // B,)

    feat_specs = [
        pl.BlockSpec((B, f.shape[1]), lambda i: (i, 0)) for f in (f0, f1, f2)
    ]
    w_specs = [pl.BlockSpec(w.shape, lambda i: (0, 0)) for w in Ws]
    b_specs = [pl.BlockSpec(bpacked.shape, lambda i: (0, 0))]

    out_widths = [(2 * L + 1, _K_MAX_L[L]) for (L, S) in _FINAL_ORDER]
    out_shape = [jax.ShapeDtypeStruct((m, n, k), jnp.float32)
                 for (m, k) in out_widths]
    out_specs = [pl.BlockSpec((m, B, k), lambda i: (0, i, 0))
                 for (m, k) in out_widths]

    res = pl.pallas_call(
        _body,
        grid=grid,
        in_specs=feat_specs + w_specs + b_specs,
        out_specs=out_specs,
        out_shape=out_shape,
        compiler_params=pltpu.CompilerParams(
            dimension_semantics=("parallel",)),
    )(f0, f1, f2, *Ws, bpacked)

    return tuple(jnp.transpose(r, (1, 0, 2))
                 for r, (m, k) in zip(res, out_widths))


# speedup vs baseline: 1.2020x; 1.2020x over previous
import numpy as np
import jax
import jax.numpy as jnp
from jax.experimental import pallas as pl
from jax.experimental.pallas import tpu as pltpu

_K_MAX_L = [128, 64, 32]
_L_MAX = 2
_IRREPS_IN = [(0, 1), (1, 1), (2, 1)]
_N_ITER = 2


def _cg_tables(l_max):
    cgs = {}
    for l1 in range(l_max + 1):
        for l2 in range(l_max + 1):
            entries = []
            norms = []
            off = 0
            for L in range(abs(l1 - l2), min(l1 + l2, l_max) + 1):
                norms.append(
                    float(1.0 / np.sqrt((2 * l1 + 1) * (2 * l2 + 1) * (2 * L + 1))))
                for M in range(2 * L + 1):
                    MU = M - L
                    for m1 in range(2 * l1 + 1):
                        mu1 = m1 - l1
                        mu2 = MU - mu1
                        if -l2 <= mu2 <= l2:
                            m2 = mu2 + l2
                            entries.append((m1, m2, off + M, (-1) ** mu1))
                off += 2 * L + 1
            cgs[(l1, l2)] = (entries, off, norms)
    return cgs


_CGS = _cg_tables(_L_MAX)


def _sim_structure():
    cur = list(_IRREPS_IN)
    orders = []
    for _ in range(_N_ITER):
        order = []
        for (l1, s1) in cur:
            for (l2, s2) in cur:
                for L in range(abs(l1 - l2), min(l1 + l2, _L_MAX) + 1):
                    S = s1 * s2 * (-1) ** (l1 + l2 + L)
                    if (L, S) not in order:
                        order.append((L, S))
        orders.append(order)
        cur = order
    return orders


_ORDERS = _sim_structure()
_FINAL_ORDER = _ORDERS[-1]
_W_KEYS = [(0, (0, 1)), (0, (1, -1)), (0, (1, 1)), (0, (2, -1)), (0, (2, 1)),
           (1, (0, -1)), (1, (0, 1)), (1, (1, -1)), (1, (1, 1)),
           (1, (2, -1)), (1, (2, 1))]


def _cg_apply(cur, Wd, bd):
    gscaled = {}
    for kk, (w, tlist) in cur.items():
        l = kk[0]
        gscaled[kk] = [t * (1.0 + 0.1 * (m - l)) for m, t in enumerate(tlist)]

    results = {}
    for k1, (w1, t1) in cur.items():
        l1, s1 = k1
        for k2, (w2, _) in cur.items():
            l2, s2 = k2
            t2g = gscaled[k2]
            ms = min(w1, w2)
            entries, size_out, norms = _CGS[(l1, l2)]
            res = [None] * size_out

            def _accum(io, p, sgn):
                if res[io] is None:
                    res[io] = p if sgn > 0 else -p
                else:
                    res[io] = res[io] + p if sgn > 0 else res[io] - p

            if k1 == k2:
                byprod = {}
                for ia, ib, io, sgn in entries:
                    byprod.setdefault((ia, ib), []).append(io)
                for (ia, ib), ios in byprod.items():
                    if ia == ib:
                        p = t1[ia][:ms, :] * t2g[ia][:ms, :]
                        sgn = (-1) ** (ia - l1)
                        for io in ios:
                            _accum(io, p, sgn)
                    elif ia < ib:
                        praw = t1[ia][:ms, :] * t1[ib][:ms, :]
                        gamma = ((-1) ** (ia - l1) * (1.0 + 0.1 * (ib - l1))
                                 + (-1) ** (ib - l1) * (1.0 + 0.1 * (ia - l1)))
                        pg = praw * gamma
                        for io in ios:
                            _accum(io, pg, 1)
            else:
                p = None
                prev = None
                for ia, ib, io, sgn in sorted(entries):
                    if (ia, ib) != prev:
                        p = t1[ia][:ms, :] * t2g[ib][:ms, :]
                        prev = (ia, ib)
                    _accum(io, p, sgn)
            off = 0
            for Li, L in enumerate(
                    range(abs(l1 - l2), min(l1 + l2, _L_MAX) + 1)):
                S = s1 * s2 * (-1) ** (l1 + l2 + L)
                results.setdefault((L, S), []).append(
                    (res[off:off + 2 * L + 1], ms, norms[Li]))
                off += 2 * L + 1
    out = {}
    for (L, S), lst in results.items():
        W = Wd[(L, S)]
        b = bd[(L, S)]
        Wchs = []
        off = 0
        for (_, ms, nrm) in lst:
            Wchs.append(W[:, off:off + ms] * nrm)
            off += ms
        outm = []
        for M in range(2 * L + 1):
            acc = None
            for (arrs, ms, _), Wch in zip(lst, Wchs):
                mm = jax.lax.dot_general(
                    Wch, arrs[M], (((1,), (0,)), ((), ())),
                    preferred_element_type=jnp.float32)
                acc = mm if acc is None else acc + mm
            outm.append(acc + b)
        out[(L, S)] = (_K_MAX_L[L], outm)
    return out


def _body(f0_ref, f1_ref, f2_ref, *refs):
    Wrefs = refs[:11]
    bref = refs[11]
    orefs = refs[12:]
    Wd = [{}, {}]
    bd = [{}, {}]
    boff = 0
    for (it, ls), wr in zip(_W_KEYS, Wrefs):
        Wd[it][ls] = wr[...]
        k = _K_MAX_L[ls[0]]
        bd[it][ls] = bref[boff:boff + k, :]
        boff += k

    f0 = f0_ref[...]
    f1 = f1_ref[...]
    f2 = f2_ref[...]
    cur = {
        (0, 1): (128, [f0]),
        (1, 1): (64, [f1[64 * m:64 * (m + 1), :] for m in range(3)]),
        (2, 1): (32, [f2[32 * m:32 * (m + 1), :] for m in range(5)]),
    }
    for it in range(_N_ITER):
        out = _cg_apply(cur, Wd[it], bd[it])
        new = {}
        for kk, (w, arrs) in out.items():
            if kk in cur and cur[kk][0] == w and len(cur[kk][1]) == len(arrs):
                new[kk] = (w, [a + b for a, b in zip(arrs, cur[kk][1])])
            else:
                new[kk] = (w, arrs)
        cur = new

    for oref, kk in zip(orefs, _FINAL_ORDER):
        w, arrs = cur[kk]
        for M, a in enumerate(arrs):
            oref[:, w * M:w * (M + 1)] = a.T


def kernel(feat_0_1, feat_1_1, feat_2_1,
           W_it0_0_p1, W_it0_1_m1, W_it0_1_p1, W_it0_2_m1, W_it0_2_p1,
           W_it1_0_m1, W_it1_0_p1, W_it1_1_m1, W_it1_1_p1, W_it1_2_m1,
           W_it1_2_p1,
           b_it0_0_p1, b_it0_1_m1, b_it0_1_p1, b_it0_2_m1, b_it0_2_p1,
           b_it1_0_m1, b_it1_0_p1, b_it1_1_m1, b_it1_1_p1, b_it1_2_m1,
           b_it1_2_p1):
    n = feat_0_1.shape[0]
    f0 = feat_0_1.reshape(n, 128).T
    f1 = feat_1_1.reshape(n, 3 * 64).T
    f2 = feat_2_1.reshape(n, 5 * 32).T

    Wmap = {
        (0, (0, 1)): W_it0_0_p1, (0, (1, -1)): W_it0_1_m1,
        (0, (1, 1)): W_it0_1_p1, (0, (2, -1)): W_it0_2_m1,
        (0, (2, 1)): W_it0_2_p1,
        (1, (0, -1)): W_it1_0_m1, (1, (0, 1)): W_it1_0_p1,
        (1, (1, -1)): W_it1_1_m1, (1, (1, 1)): W_it1_1_p1,
        (1, (2, -1)): W_it1_2_m1, (1, (2, 1)): W_it1_2_p1,
    }
    bmap = {
        (0, (0, 1)): b_it0_0_p1, (0, (1, -1)): b_it0_1_m1,
        (0, (1, 1)): b_it0_1_p1, (0, (2, -1)): b_it0_2_m1,
        (0, (2, 1)): b_it0_2_p1,
        (1, (0, -1)): b_it1_0_m1, (1, (0, 1)): b_it1_0_p1,
        (1, (1, -1)): b_it1_1_m1, (1, (1, 1)): b_it1_1_p1,
        (1, (2, -1)): b_it1_2_m1, (1, (2, 1)): b_it1_2_p1,
    }
    Ws = [Wmap[k] for k in _W_KEYS]
    bpacked = jnp.concatenate([bmap[k] for k in _W_KEYS]).reshape(-1, 1)

    B = min(1024, n)
    grid = (n // B,)

    feat_specs = [
        pl.BlockSpec((f.shape[0], B), lambda i: (0, i)) for f in (f0, f1, f2)
    ]
    w_specs = [pl.BlockSpec(w.shape, lambda i: (0, 0)) for w in Ws]
    b_specs = [pl.BlockSpec(bpacked.shape, lambda i: (0, 0))]

    out_widths = [(2 * L + 1, _K_MAX_L[L]) for (L, S) in _FINAL_ORDER]
    out_shape = [jax.ShapeDtypeStruct((n, m * k), jnp.float32)
                 for (m, k) in out_widths]
    out_specs = [pl.BlockSpec((B, m * k), lambda i: (i, 0))
                 for (m, k) in out_widths]

    res = pl.pallas_call(
        _body,
        grid=grid,
        in_specs=feat_specs + w_specs + b_specs,
        out_specs=out_specs,
        out_shape=out_shape,
        compiler_params=pltpu.CompilerParams(
            dimension_semantics=("parallel",)),
    )(f0, f1, f2, *Ws, bpacked)

    return tuple(r.reshape(n, m, k)
                 for r, (m, k) in zip(res, out_widths))


# speedup vs baseline: 1.4937x; 1.2427x over previous
import numpy as np
import jax
import jax.numpy as jnp
from jax.experimental import pallas as pl
from jax.experimental.pallas import tpu as pltpu

_K_MAX_L = [128, 64, 32]
_L_MAX = 2
_IRREPS_IN = [(0, 1), (1, 1), (2, 1)]
_N_ITER = 2


def _cg_tables(l_max):
    cgs = {}
    for l1 in range(l_max + 1):
        for l2 in range(l_max + 1):
            entries = []
            norms = []
            off = 0
            for L in range(abs(l1 - l2), min(l1 + l2, l_max) + 1):
                norms.append(
                    float(1.0 / np.sqrt((2 * l1 + 1) * (2 * l2 + 1) * (2 * L + 1))))
                for M in range(2 * L + 1):
                    MU = M - L
                    for m1 in range(2 * l1 + 1):
                        mu1 = m1 - l1
                        mu2 = MU - mu1
                        if -l2 <= mu2 <= l2:
                            m2 = mu2 + l2
                            entries.append((m1, m2, off + M, (-1) ** mu1))
                off += 2 * L + 1
            cgs[(l1, l2)] = (entries, off, norms)
    return cgs


_CGS = _cg_tables(_L_MAX)


def _sim_structure():
    cur = list(_IRREPS_IN)
    orders = []
    for _ in range(_N_ITER):
        order = []
        for (l1, s1) in cur:
            for (l2, s2) in cur:
                for L in range(abs(l1 - l2), min(l1 + l2, _L_MAX) + 1):
                    S = s1 * s2 * (-1) ** (l1 + l2 + L)
                    if (L, S) not in order:
                        order.append((L, S))
        orders.append(order)
        cur = order
    return orders


_ORDERS = _sim_structure()
_FINAL_ORDER = _ORDERS[-1]
_W_KEYS = [(0, (0, 1)), (0, (1, -1)), (0, (1, 1)), (0, (2, -1)), (0, (2, 1)),
           (1, (0, -1)), (1, (0, 1)), (1, (1, -1)), (1, (1, 1)),
           (1, (2, -1)), (1, (2, 1))]


def _cg_apply(cur, Wd, bd):
    gscaled = {}
    for kk, (w, tlist) in cur.items():
        l = kk[0]
        gscaled[kk] = [t * (1.0 + 0.1 * (m - l)) for m, t in enumerate(tlist)]

    results = {}
    for k1, (w1, t1) in cur.items():
        l1, s1 = k1
        for k2, (w2, _) in cur.items():
            l2, s2 = k2
            t2g = gscaled[k2]
            ms = min(w1, w2)
            entries, size_out, norms = _CGS[(l1, l2)]
            res = [None] * size_out

            def _accum(io, p, sgn):
                if res[io] is None:
                    res[io] = p if sgn > 0 else -p
                else:
                    res[io] = res[io] + p if sgn > 0 else res[io] - p

            if k1 == k2:
                byprod = {}
                for ia, ib, io, sgn in entries:
                    byprod.setdefault((ia, ib), []).append(io)
                for (ia, ib), ios in byprod.items():
                    if ia == ib:
                        p = t1[ia][:ms, :] * t2g[ia][:ms, :]
                        sgn = (-1) ** (ia - l1)
                        for io in ios:
                            _accum(io, p, sgn)
                    elif ia < ib:
                        praw = t1[ia][:ms, :] * t1[ib][:ms, :]
                        gamma = ((-1) ** (ia - l1) * (1.0 + 0.1 * (ib - l1))
                                 + (-1) ** (ib - l1) * (1.0 + 0.1 * (ia - l1)))
                        pg = praw * gamma
                        for io in ios:
                            _accum(io, pg, 1)
            else:
                p = None
                prev = None
                for ia, ib, io, sgn in sorted(entries):
                    if (ia, ib) != prev:
                        p = t1[ia][:ms, :] * t2g[ib][:ms, :]
                        prev = (ia, ib)
                    _accum(io, p, sgn)
            off = 0
            for Li, L in enumerate(
                    range(abs(l1 - l2), min(l1 + l2, _L_MAX) + 1)):
                S = s1 * s2 * (-1) ** (l1 + l2 + L)
                results.setdefault((L, S), []).append(
                    (res[off:off + 2 * L + 1], ms, norms[Li]))
                off += 2 * L + 1
    out = {}
    for (L, S), lst in results.items():
        W = Wd[(L, S)]
        b = bd[(L, S)]
        Wchs = []
        off = 0
        for (_, ms, nrm) in lst:
            Wchs.append(W[:, off:off + ms] * nrm)
            off += ms
        outm = []
        for M in range(2 * L + 1):
            acc = None
            for (arrs, ms, _), Wch in zip(lst, Wchs):
                mm = jax.lax.dot_general(
                    Wch, arrs[M], (((1,), (0,)), ((), ())),
                    preferred_element_type=jnp.float32)
                acc = mm if acc is None else acc + mm
            outm.append(acc + b)
        out[(L, S)] = (_K_MAX_L[L], outm)
    return out


def _body(f0_ref, f1_ref, f2_ref, *refs):
    Wrefs = refs[:11]
    bref = refs[11]
    orefs = refs[12:]
    Wd = [{}, {}]
    bd = [{}, {}]
    boff = 0
    for (it, ls), wr in zip(_W_KEYS, Wrefs):
        Wd[it][ls] = wr[...]
        k = _K_MAX_L[ls[0]]
        bd[it][ls] = bref[boff:boff + k, :]
        boff += k

    f0 = f0_ref[...]
    f1 = f1_ref[...]
    f2 = f2_ref[...]
    cur = {
        (0, 1): (128, [f0]),
        (1, 1): (64, [f1[64 * m:64 * (m + 1), :] for m in range(3)]),
        (2, 1): (32, [f2[32 * m:32 * (m + 1), :] for m in range(5)]),
    }
    for it in range(_N_ITER):
        out = _cg_apply(cur, Wd[it], bd[it])
        new = {}
        for kk, (w, arrs) in out.items():
            if kk in cur and cur[kk][0] == w and len(cur[kk][1]) == len(arrs):
                new[kk] = (w, [a + b for a, b in zip(arrs, cur[kk][1])])
            else:
                new[kk] = (w, arrs)
        cur = new

    for oref, kk in zip(orefs, _FINAL_ORDER):
        w, arrs = cur[kk]
        for M, a in enumerate(arrs):
            oref[w * M:w * (M + 1), :] = a


def kernel(feat_0_1, feat_1_1, feat_2_1,
           W_it0_0_p1, W_it0_1_m1, W_it0_1_p1, W_it0_2_m1, W_it0_2_p1,
           W_it1_0_m1, W_it1_0_p1, W_it1_1_m1, W_it1_1_p1, W_it1_2_m1,
           W_it1_2_p1,
           b_it0_0_p1, b_it0_1_m1, b_it0_1_p1, b_it0_2_m1, b_it0_2_p1,
           b_it1_0_m1, b_it1_0_p1, b_it1_1_m1, b_it1_1_p1, b_it1_2_m1,
           b_it1_2_p1):
    n = feat_0_1.shape[0]
    f0 = feat_0_1.reshape(n, 128).T
    f1 = feat_1_1.reshape(n, 3 * 64).T
    f2 = feat_2_1.reshape(n, 5 * 32).T

    Wmap = {
        (0, (0, 1)): W_it0_0_p1, (0, (1, -1)): W_it0_1_m1,
        (0, (1, 1)): W_it0_1_p1, (0, (2, -1)): W_it0_2_m1,
        (0, (2, 1)): W_it0_2_p1,
        (1, (0, -1)): W_it1_0_m1, (1, (0, 1)): W_it1_0_p1,
        (1, (1, -1)): W_it1_1_m1, (1, (1, 1)): W_it1_1_p1,
        (1, (2, -1)): W_it1_2_m1, (1, (2, 1)): W_it1_2_p1,
    }
    bmap = {
        (0, (0, 1)): b_it0_0_p1, (0, (1, -1)): b_it0_1_m1,
        (0, (1, 1)): b_it0_1_p1, (0, (2, -1)): b_it0_2_m1,
        (0, (2, 1)): b_it0_2_p1,
        (1, (0, -1)): b_it1_0_m1, (1, (0, 1)): b_it1_0_p1,
        (1, (1, -1)): b_it1_1_m1, (1, (1, 1)): b_it1_1_p1,
        (1, (2, -1)): b_it1_2_m1, (1, (2, 1)): b_it1_2_p1,
    }
    Ws = [Wmap[k] for k in _W_KEYS]
    bpacked = jnp.concatenate([bmap[k] for k in _W_KEYS]).reshape(-1, 1)

    B = min(1024, n)
    grid = (n // B,)

    feat_specs = [
        pl.BlockSpec((f.shape[0], B), lambda i: (0, i)) for f in (f0, f1, f2)
    ]
    w_specs = [pl.BlockSpec(w.shape, lambda i: (0, 0)) for w in Ws]
    b_specs = [pl.BlockSpec(bpacked.shape, lambda i: (0, 0))]

    out_widths = [(2 * L + 1, _K_MAX_L[L]) for (L, S) in _FINAL_ORDER]
    out_shape = [jax.ShapeDtypeStruct((m * k, n), jnp.float32)
                 for (m, k) in out_widths]
    out_specs = [pl.BlockSpec((m * k, B), lambda i: (0, i))
                 for (m, k) in out_widths]

    res = pl.pallas_call(
        _body,
        grid=grid,
        in_specs=feat_specs + w_specs + b_specs,
        out_specs=out_specs,
        out_shape=out_shape,
        compiler_params=pltpu.CompilerParams(
            dimension_semantics=("parallel",)),
    )(f0, f1, f2, *Ws, bpacked)

    return tuple(r.T.reshape(n, m, k)
                 for r, (m, k) in zip(res, out_widths))
